# Initial kernel scaffold; baseline (speedup 1.0000x reference)
import jax
import jax.numpy as jnp
from jax.experimental import pallas as pl


def kernel(x, planes_weight):
    raise NotImplementedError("write your pallas kernel here")


# speedup vs baseline: 1.1866x; 1.1866x over previous
import functools

import jax
import jax.numpy as jnp
from jax import lax
from jax.experimental import pallas as pl
from jax.experimental.pallas import tpu as pltpu
from jax.experimental.pallas import tpu_sc as plsc

_L = 128
_NW = 32


@functools.lru_cache(maxsize=None)
def _build_gather(n_groups, g_per_w, dim):
    mesh = plsc.VectorSubcoreMesh(core_axis_name="c", subcore_axis_name="s")

    @functools.partial(
        pl.kernel,
        mesh=mesh,
        out_type=jax.ShapeDtypeStruct((n_groups, _L, dim), jnp.float32),
        scratch_types=[
            pltpu.VMEM((g_per_w, _L), jnp.int32),
            pltpu.VMEM((_L, dim), jnp.float32),
            pltpu.SemaphoreType.DMA,
        ],
        compiler_params=pltpu.CompilerParams(use_tc_tiling_on_sc=False),
    )
    def gather_kernel(idx_hbm, table_hbm, out_hbm, idx_v, rows_v, sem):
        wid = lax.axis_index("s") * 2 + lax.axis_index("c")
        gbase = wid * g_per_w
        pltpu.sync_copy(idx_hbm.at[pl.ds(gbase, g_per_w)], idx_v)

        def body(g, carry):
            pltpu.async_copy(table_hbm.at[idx_v.at[g]], rows_v, sem).wait()
            pltpu.sync_copy(rows_v, out_hbm.at[gbase + g])
            return carry

        lax.fori_loop(0, g_per_w, body, 0)

    return gather_kernel


def kernel(x, planes_weight):
    b, s = x.shape
    _, dim = planes_weight.shape
    n = b * s
    n_groups = n // _L
    g_per_w = n_groups // _NW
    idx = x.reshape(n_groups, _L).astype(jnp.int32)
    out = _build_gather(n_groups, g_per_w, dim)(idx, planes_weight)
    return out.reshape(b, s, dim)
